# baseline (speedup 1.0000x reference)
import jax
import jax.numpy as jnp
from jax.experimental import pallas as pl

_BT = 2048


def _vq_block(x_ref, cb_ref, out_ref):
    xb = x_ref[...]
    cb = cb_ref[...]
    scores = jax.lax.dot_general(
        xb, cb, (((1,), (1,)), ((), ())),
        preferred_element_type=jnp.float32)
    c2 = jnp.sum(cb * cb, axis=1)
    x2 = jnp.sum(xb * xb, axis=1, keepdims=True)
    dist = (x2 + c2[None, :]) - 2.0 * scores
    idx = jnp.argmin(dist, axis=1)
    out_ref[0, 0, :] = idx.astype(jnp.int32)


def kernel(x, codebook):
    B, T, D = x.shape
    K = codebook.shape[0]
    flat = x.reshape(B * T, D)
    grid = (B * T) // _BT
    out = pl.pallas_call(
        _vq_block,
        grid=(grid,),
        in_specs=[
            pl.BlockSpec((_BT, D), lambda i: (i, 0)),
            pl.BlockSpec((K, D), lambda i: (0, 0)),
        ],
        out_specs=pl.BlockSpec((1, 1, _BT), lambda i: (i, 0, 0)),
        out_shape=jax.ShapeDtypeStruct((grid, 1, _BT), jnp.int32),
    )(flat, codebook)
    return out.reshape(B, T)


# speedup vs baseline: 1.0996x; 1.0996x over previous
import jax
import jax.numpy as jnp
from jax.experimental import pallas as pl

_BT = 1024


def _vq_block(x_ref, cb_ref, out_ref):
    xb = x_ref[...]
    cb = cb_ref[...]
    scores = jax.lax.dot_general(
        xb, cb, (((1,), (1,)), ((), ())),
        preferred_element_type=jnp.float32)
    c2 = jnp.sum(cb * cb, axis=1)
    x2 = jnp.sum(xb * xb, axis=1, keepdims=True)
    dist = (x2 + c2[None, :]) - 2.0 * scores
    idx = jnp.argmin(dist, axis=1)
    out_ref[0, 0, :] = idx.astype(jnp.int32)


def kernel(x, codebook):
    B, T, D = x.shape
    K = codebook.shape[0]
    flat = x.reshape(B * T, D)
    grid = (B * T) // _BT
    out = pl.pallas_call(
        _vq_block,
        grid=(grid,),
        in_specs=[
            pl.BlockSpec((_BT, D), lambda i: (i, 0)),
            pl.BlockSpec((K, D), lambda i: (0, 0)),
        ],
        out_specs=pl.BlockSpec((1, 1, _BT), lambda i: (i, 0, 0)),
        out_shape=jax.ShapeDtypeStruct((grid, 1, _BT), jnp.int32),
    )(flat, codebook)
    return out.reshape(B, T)
